# baseline (speedup 1.0000x reference)
import functools

import jax
import jax.numpy as jnp
from jax import lax
from jax.experimental import pallas as pl
from jax.experimental.pallas import tpu as pltpu
from jax.experimental.pallas import tpu_sc as plsc

_POS_OFFSET = 2


@functools.lru_cache(maxsize=None)
def _make_sc_lookup(B, S, D, dtype):
    info = plsc.get_sparse_core_info()
    num_workers = info.num_cores * info.num_subcores
    rows_per_w = S // num_workers
    rb = 32
    while rows_per_w % rb:
        rb //= 2
    n_chunks = rows_per_w // rb
    mesh = plsc.VectorSubcoreMesh(core_axis_name="c", subcore_axis_name="s")

    def body(table_hbm, out_hbm, buf0, buf1, rsem0, rsem1, wsem0, wsem1):
        wid = lax.axis_index("s") * info.num_cores + lax.axis_index("c")
        base = wid * rows_per_w
        bufs, rsems, wsems = (buf0, buf1), (rsem0, rsem1), (wsem0, wsem1)
        pending_writes = {0: [], 1: []}
        reads = {}

        def start_read(j):
            r0 = base + j * rb
            reads[j] = pltpu.async_copy(
                table_hbm.at[pl.ds(r0 + _POS_OFFSET, rb), :],
                bufs[j % 2], rsems[j % 2])

        start_read(0)
        for j in range(n_chunks):
            p = j % 2
            reads[j].wait()
            r0 = base + j * rb
            for b in range(B):
                pending_writes[p].append(pltpu.async_copy(
                    bufs[p], out_hbm.at[b, pl.ds(r0, rb), :], wsems[p]))
            if j + 1 < n_chunks:
                q = (j + 1) % 2
                for w in pending_writes[q]:
                    w.wait()
                pending_writes[q] = []
                start_read(j + 1)
        for p in (0, 1):
            for w in pending_writes[p]:
                w.wait()

    return pl.kernel(
        body,
        out_type=jax.ShapeDtypeStruct((B, S, D), dtype),
        mesh=mesh,
        scratch_types=[
            pltpu.VMEM((rb, D), dtype),
            pltpu.VMEM((rb, D), dtype),
            pltpu.SemaphoreType.DMA,
            pltpu.SemaphoreType.DMA,
            pltpu.SemaphoreType.DMA,
            pltpu.SemaphoreType.DMA,
        ],
        compiler_params=pltpu.CompilerParams(use_tc_tiling_on_sc=False),
    )


@jax.jit
def kernel(inputs_embeds, table):
    B, S, _ = inputs_embeds.shape
    D = table.shape[1]
    return _make_sc_lookup(B, S, D, table.dtype)(table)


# speedup vs baseline: 2.6140x; 2.6140x over previous
import functools

import jax
import jax.numpy as jnp
from jax import lax
from jax.experimental import pallas as pl
from jax.experimental.pallas import tpu as pltpu
from jax.experimental.pallas import tpu_sc as plsc

_POS_OFFSET = 2


@functools.lru_cache(maxsize=None)
def _make_sc_lookup(B, S, D, dtype):
    info = plsc.get_sparse_core_info()
    num_workers = info.num_cores * info.num_subcores
    L = info.num_lanes
    rows_per_w = S // num_workers
    rb = L
    n_chunks = rows_per_w // rb
    mesh = plsc.VectorSubcoreMesh(core_axis_name="c", subcore_axis_name="s")

    def body(table_hbm, out_hbm, buf0, buf1, rsem0, rsem1, wsem0, wsem1):
        wid = lax.axis_index("s") * info.num_cores + lax.axis_index("c")
        base = wid * rows_per_w
        bufs, rsems, wsems = (buf0, buf1), (rsem0, rsem1), (wsem0, wsem1)
        pending_writes = {0: [], 1: []}
        reads = {}
        lane = lax.iota(jnp.int32, L)

        def start_read(j):
            idx = lane + (base + j * rb + _POS_OFFSET)
            reads[j] = pltpu.async_copy(
                table_hbm.at[idx], bufs[j % 2], rsems[j % 2])

        start_read(0)
        for j in range(n_chunks):
            p = j % 2
            reads[j].wait()
            r0 = base + j * rb
            for b in range(B):
                pending_writes[p].append(pltpu.async_copy(
                    bufs[p], out_hbm.at[b, pl.ds(r0, rb), :], wsems[p]))
            if j + 1 < n_chunks:
                q = (j + 1) % 2
                for w in pending_writes[q]:
                    w.wait()
                pending_writes[q] = []
                start_read(j + 1)
        for p in (0, 1):
            for w in pending_writes[p]:
                w.wait()

    return pl.kernel(
        body,
        out_type=jax.ShapeDtypeStruct((B, S, D), dtype),
        mesh=mesh,
        scratch_types=[
            pltpu.VMEM((rb, D), dtype),
            pltpu.VMEM((rb, D), dtype),
            pltpu.SemaphoreType.DMA,
            pltpu.SemaphoreType.DMA,
            pltpu.SemaphoreType.DMA,
            pltpu.SemaphoreType.DMA,
        ],
    )


@jax.jit
def kernel(inputs_embeds, table):
    B, S, _ = inputs_embeds.shape
    D = table.shape[1]
    return _make_sc_lookup(B, S, D, table.dtype)(table)


# speedup vs baseline: 2.6837x; 1.0267x over previous
import functools

import jax
import jax.numpy as jnp
from jax import lax
from jax.experimental import pallas as pl
from jax.experimental.pallas import tpu as pltpu
from jax.experimental.pallas import tpu_sc as plsc

_POS_OFFSET = 2


@functools.lru_cache(maxsize=None)
def _make_sc_lookup(B, S, D, dtype):
    info = plsc.get_sparse_core_info()
    num_workers = info.num_cores * info.num_subcores
    L = info.num_lanes
    rows_per_w = S // num_workers
    rb = L
    n_chunks = rows_per_w // rb
    mesh = plsc.VectorSubcoreMesh(core_axis_name="c", subcore_axis_name="s")

    nbuf = 4

    def body(table_hbm, out_hbm, *scratch):
        bufs = scratch[:nbuf]
        rsems = scratch[nbuf:2 * nbuf]
        wsems = scratch[2 * nbuf:3 * nbuf]
        wid = lax.axis_index("s") * info.num_cores + lax.axis_index("c")
        base = wid * rows_per_w
        pending_writes = {p: [] for p in range(nbuf)}
        reads = {}
        lane = lax.iota(jnp.int32, L)

        def start_read(j):
            idx = lane + (base + j * rb + _POS_OFFSET)
            reads[j] = pltpu.async_copy(
                table_hbm.at[idx], bufs[j % nbuf], rsems[j % nbuf])

        for j in range(min(nbuf, n_chunks)):
            start_read(j)
        for j in range(n_chunks):
            p = j % nbuf
            reads[j].wait()
            r0 = base + j * rb
            for b in range(B):
                pending_writes[p].append(pltpu.async_copy(
                    bufs[p], out_hbm.at[b, pl.ds(r0, rb), :], wsems[p]))
            nxt = j + nbuf
            if nxt < n_chunks:
                q = nxt % nbuf
                for w in pending_writes[q]:
                    w.wait()
                pending_writes[q] = []
                start_read(nxt)
        for p in range(nbuf):
            for w in pending_writes[p]:
                w.wait()

    return pl.kernel(
        body,
        out_type=jax.ShapeDtypeStruct((B, S, D), dtype),
        mesh=mesh,
        scratch_types=(
            [pltpu.VMEM((rb, D), dtype)] * nbuf
            + [pltpu.SemaphoreType.DMA] * (2 * nbuf)
        ),
    )


@jax.jit
def kernel(inputs_embeds, table):
    B, S, _ = inputs_embeds.shape
    D = table.shape[1]
    return _make_sc_lookup(B, S, D, table.dtype)(table)


# speedup vs baseline: 2.8429x; 1.0593x over previous
import functools

import jax
import jax.numpy as jnp
from jax import lax
from jax.experimental import pallas as pl
from jax.experimental.pallas import tpu as pltpu
from jax.experimental.pallas import tpu_sc as plsc

_POS_OFFSET = 2


@functools.lru_cache(maxsize=None)
def _make_sc_lookup(B, S, D, dtype):
    info = plsc.get_sparse_core_info()
    num_workers = info.num_cores * info.num_subcores
    L = info.num_lanes
    rows_per_w = S // num_workers
    rb = 2 * L
    n_chunks = rows_per_w // rb
    mesh = plsc.VectorSubcoreMesh(core_axis_name="c", subcore_axis_name="s")

    nbuf = 3

    def body(table_hbm, out_hbm, *scratch):
        bufs = scratch[:nbuf]
        idxs = scratch[nbuf:2 * nbuf]
        rsems = scratch[2 * nbuf:3 * nbuf]
        wsems = scratch[3 * nbuf:4 * nbuf]
        wid = lax.axis_index("s") * info.num_cores + lax.axis_index("c")
        base = wid * rows_per_w
        pending_writes = {p: [] for p in range(nbuf)}
        reads = {}
        lane = lax.iota(jnp.int32, L)

        def start_read(j):
            p = j % nbuf
            r2 = base + j * rb + _POS_OFFSET
            for v in range(rb // L):
                idxs[p][pl.ds(v * L, L)] = lane + (r2 + v * L)
            reads[j] = pltpu.async_copy(
                table_hbm.at[idxs[p]], bufs[p], rsems[p])

        for j in range(min(nbuf, n_chunks)):
            start_read(j)
        for j in range(n_chunks):
            p = j % nbuf
            reads[j].wait()
            r0 = base + j * rb
            for b in range(B):
                pending_writes[p].append(pltpu.async_copy(
                    bufs[p], out_hbm.at[b, pl.ds(r0, rb), :], wsems[p]))
            nxt = j + nbuf
            if nxt < n_chunks:
                q = nxt % nbuf
                for w in pending_writes[q]:
                    w.wait()
                pending_writes[q] = []
                start_read(nxt)
        for p in range(nbuf):
            for w in pending_writes[p]:
                w.wait()

    return pl.kernel(
        body,
        out_type=jax.ShapeDtypeStruct((B, S, D), dtype),
        mesh=mesh,
        scratch_types=(
            [pltpu.VMEM((rb, D), dtype)] * nbuf
            + [pltpu.VMEM((rb,), jnp.int32)] * nbuf
            + [pltpu.SemaphoreType.DMA] * (2 * nbuf)
        ),
    )


@jax.jit
def kernel(inputs_embeds, table):
    B, S, _ = inputs_embeds.shape
    D = table.shape[1]
    return _make_sc_lookup(B, S, D, table.dtype)(table)
